# baseline (speedup 1.0000x reference)
import jax
import jax.numpy as jnp
from jax.experimental import pallas as pl

def _body(x_ref, i_ref, o_ref):
    o_ref[...] = x_ref[...] + i_ref[...].astype(jnp.float32)

def kernel(input_xyzs, query_xyz_index):
    x = input_xyzs.reshape(8, 24576)
    i = query_xyz_index.reshape(8, 24576)
    out = pl.pallas_call(
        _body,
        out_shape=jax.ShapeDtypeStruct((8, 24576), jnp.float32),
    )(x, i)
    return out.reshape(65536, 3)


# speedup vs baseline: 70.7441x; 70.7441x over previous
import jax
import jax.numpy as jnp
from jax.experimental import pallas as pl

def _body(x_ref, i_ref, o_ref):
    o_ref[...] = x_ref[...] + i_ref[...].astype(jnp.float32)

def kernel(input_xyzs, query_xyz_index):
    x = input_xyzs.T
    i = query_xyz_index.T
    out = pl.pallas_call(
        _body,
        out_shape=jax.ShapeDtypeStruct((3, 65536), jnp.float32),
    )(x, i)
    return out.T
